# baseline (speedup 1.0000x reference)
import functools

import jax
import jax.numpy as jnp
from jax import lax
from jax.experimental import pallas as pl
from jax.experimental.pallas import tpu as pltpu
from jax.experimental.pallas import tpu_sc as plsc

MAX_LEN = 8192
EMB = 64

NC = 2
NS = 16
NW = NC * NS
LANES = 16


def _fuse_body(w_ref, p_ref, o_ref):
    o_ref[...] = w_ref[...] + p_ref[...]


def _fuse_tables(word_head, pos_table):
    return pl.pallas_call(
        _fuse_body,
        out_shape=jax.ShapeDtypeStruct((MAX_LEN, EMB), jnp.float32),
    )(word_head, pos_table)


def _gather_kernel(B, T):
    assert B % NW == 0
    SLAB = B // NW
    assert SLAB % LANES == 0
    NJ = SLAB // LANES
    assert T % 2 == 0
    pairs = T // 2

    mesh = plsc.VectorSubcoreMesh(core_axis_name="c", subcore_axis_name="s")

    @functools.partial(
        pl.kernel,
        out_type=jax.ShapeDtypeStruct((T, EMB, B), jnp.float32),
        mesh=mesh,
        scratch_types=[
            pltpu.VMEM((SLAB, T), jnp.int32),
            pltpu.VMEM((T, SLAB), jnp.int32),
            pltpu.VMEM((SLAB, EMB), jnp.float32),
            pltpu.VMEM((SLAB, EMB), jnp.float32),
            pltpu.VMEM((EMB, SLAB), jnp.float32),
            pltpu.VMEM((EMB, SLAB), jnp.float32),
            pltpu.SemaphoreType.DMA,
            pltpu.SemaphoreType.DMA,
            pltpu.SemaphoreType.DMA,
            pltpu.SemaphoreType.DMA,
        ],
        compiler_params=pltpu.CompilerParams(use_tc_tiling_on_sc=False,
                                             needs_layout_passes=False),
    )
    def k(idx_hbm, table_hbm, out_hbm, idx_raw, idx_t, gbuf_a, gbuf_b,
          tbuf_a, tbuf_b, gsem_a, gsem_b, osem_a, osem_b):
        wid = lax.axis_index("s") * NC + lax.axis_index("c")
        b0 = wid * SLAB
        pltpu.sync_copy(idx_hbm.at[pl.ds(b0, SLAB)], idx_raw)

        iota = lax.iota(jnp.int32, LANES)
        rows = [iota + (LANES * j) for j in range(NJ)]

        def idx_transpose(t, _):
            tv = jnp.full((LANES,), t, jnp.int32)
            for j in range(NJ):
                v = plsc.load_gather(idx_raw, [rows[j], tv])
                plsc.store_scatter(idx_t, [tv, rows[j]], v)
            return 0

        lax.fori_loop(0, T, idx_transpose, 0)

        def fire_gather(gbuf, t, sem):
            pltpu.async_copy(table_hbm.at[idx_t.at[t]], gbuf, sem)

        def wait_gather(gbuf, sem):
            pltpu.make_async_copy(table_hbm.at[pl.ds(0, SLAB)], gbuf,
                                  sem).wait()

        def transpose(gbuf, tbuf):
            for e in range(EMB):
                ev = jnp.full((LANES,), e, jnp.int32)
                for j in range(NJ):
                    v = plsc.load_gather(gbuf, [rows[j], ev])
                    plsc.store_scatter(tbuf, [ev, rows[j]], v)

        def out_copy(tbuf, t, sem):
            return pltpu.make_async_copy(
                tbuf, out_hbm.at[t, :, pl.ds(b0, SLAB)], sem)

        fire_gather(gbuf_a, 0, gsem_a)
        fire_gather(gbuf_b, 1, gsem_b)

        def pair(i, _):
            t_a = 2 * i
            t_b = 2 * i + 1

            wait_gather(gbuf_a, gsem_a)

            @pl.when(i > 0)
            def _():
                out_copy(tbuf_a, t_a - 2, osem_a).wait()

            transpose(gbuf_a, tbuf_a)
            out_copy(tbuf_a, t_a, osem_a).start()

            @pl.when(i < pairs - 1)
            def _():
                fire_gather(gbuf_a, t_a + 2, gsem_a)

            wait_gather(gbuf_b, gsem_b)

            @pl.when(i > 0)
            def _():
                out_copy(tbuf_b, t_b - 2, osem_b).wait()

            transpose(gbuf_b, tbuf_b)
            out_copy(tbuf_b, t_b, osem_b).start()

            @pl.when(i < pairs - 1)
            def _():
                fire_gather(gbuf_b, t_b + 2, gsem_b)

            return 0

        lax.fori_loop(0, pairs, pair, 0)
        out_copy(tbuf_a, T - 2, osem_a).wait()
        out_copy(tbuf_b, T - 1, osem_b).wait()

    return k


def kernel(X, word_table, pos_table):
    B, T = X.shape
    fused = _fuse_tables(word_table[:MAX_LEN], pos_table)
    out_t = _gather_kernel(B, T)(X, fused)
    return jnp.transpose(out_t, (2, 0, 1))


# speedup vs baseline: 1.3445x; 1.3445x over previous
import functools

import jax
import jax.numpy as jnp
from jax import lax
from jax.experimental import pallas as pl
from jax.experimental.pallas import tpu as pltpu
from jax.experimental.pallas import tpu_sc as plsc

MAX_LEN = 8192
EMB = 64

NC = 2
NS = 16
NW = NC * NS
LANES = 16


def _fuse_body(w_ref, p_ref, o_ref):
    o_ref[...] = w_ref[...] + p_ref[...]


def _fuse_tables(word_head, pos_table):
    return pl.pallas_call(
        _fuse_body,
        out_shape=jax.ShapeDtypeStruct((MAX_LEN, EMB), jnp.float32),
    )(word_head, pos_table)


def _gather_kernel(B, T):
    assert B % NW == 0
    SLAB = B // NW
    assert SLAB % LANES == 0
    NJ = SLAB // LANES
    assert T % 2 == 0
    pairs = T // 2

    mesh = plsc.VectorSubcoreMesh(core_axis_name="c", subcore_axis_name="s")

    @functools.partial(
        pl.kernel,
        out_type=jax.ShapeDtypeStruct((T, EMB, B), jnp.float32),
        mesh=mesh,
        scratch_types=[
            pltpu.VMEM((SLAB, T), jnp.int32),
            pltpu.VMEM((T, SLAB), jnp.int32),
            pltpu.VMEM((SLAB, EMB), jnp.float32),
            pltpu.VMEM((SLAB, EMB), jnp.float32),
            pltpu.VMEM((EMB, SLAB), jnp.float32),
            pltpu.VMEM((EMB, SLAB), jnp.float32),
            pltpu.SemaphoreType.DMA,
            pltpu.SemaphoreType.DMA,
            pltpu.SemaphoreType.DMA,
            pltpu.SemaphoreType.DMA,
        ],
        compiler_params=pltpu.CompilerParams(use_tc_tiling_on_sc=False,
                                             needs_layout_passes=False),
    )
    def k(idx_hbm, table_hbm, out_hbm, idx_raw, idx_t, gbuf_a, gbuf_b,
          tbuf_a, tbuf_b, gsem_a, gsem_b, osem_a, osem_b):
        wid = lax.axis_index("s") * NC + lax.axis_index("c")
        b0 = wid * SLAB
        pltpu.sync_copy(idx_hbm.at[pl.ds(b0, SLAB)], idx_raw)

        iota = lax.iota(jnp.int32, LANES)
        rows = [iota + (LANES * j) for j in range(NJ)]

        def idx_transpose(t, _):
            tv = jnp.full((LANES,), t, jnp.int32)
            for j in range(NJ):
                v = plsc.load_gather(idx_raw, [rows[j], tv])
                plsc.store_scatter(idx_t, [tv, rows[j]], v)
            return 0

        lax.fori_loop(0, T, idx_transpose, 0)

        def fire_gather(gbuf, t, sem):
            pltpu.async_copy(table_hbm.at[idx_t.at[t]], gbuf, sem)

        def wait_gather(gbuf, sem):
            pltpu.make_async_copy(table_hbm.at[pl.ds(0, SLAB)], gbuf,
                                  sem).wait()

        def transpose(gbuf, tbuf):
            for e in range(EMB):
                ev = jnp.full((LANES,), e, jnp.int32)
                vals = [plsc.load_gather(gbuf, [rows[j], ev])
                        for j in range(NJ)]
                for j in range(NJ):
                    tbuf[e, pl.ds(LANES * j, LANES)] = vals[j]

        def out_copy(tbuf, t, sem):
            return pltpu.make_async_copy(
                tbuf, out_hbm.at[t, :, pl.ds(b0, SLAB)], sem)

        fire_gather(gbuf_a, 0, gsem_a)
        fire_gather(gbuf_b, 1, gsem_b)

        def pair(i, _):
            t_a = 2 * i
            t_b = 2 * i + 1

            wait_gather(gbuf_a, gsem_a)

            @pl.when(i > 0)
            def _():
                out_copy(tbuf_a, t_a - 2, osem_a).wait()

            transpose(gbuf_a, tbuf_a)
            out_copy(tbuf_a, t_a, osem_a).start()

            @pl.when(i < pairs - 1)
            def _():
                fire_gather(gbuf_a, t_a + 2, gsem_a)

            wait_gather(gbuf_b, gsem_b)

            @pl.when(i > 0)
            def _():
                out_copy(tbuf_b, t_b - 2, osem_b).wait()

            transpose(gbuf_b, tbuf_b)
            out_copy(tbuf_b, t_b, osem_b).start()

            @pl.when(i < pairs - 1)
            def _():
                fire_gather(gbuf_b, t_b + 2, gsem_b)

            return 0

        lax.fori_loop(0, pairs, pair, 0)
        out_copy(tbuf_a, T - 2, osem_a).wait()
        out_copy(tbuf_b, T - 1, osem_b).wait()

    return k


def kernel(X, word_table, pos_table):
    B, T = X.shape
    fused = _fuse_tables(word_table[:MAX_LEN], pos_table)
    out_t = _gather_kernel(B, T)(X, fused)
    return jnp.transpose(out_t, (2, 0, 1))


# speedup vs baseline: 6.0998x; 4.5367x over previous
import functools

import jax
import jax.numpy as jnp
from jax import lax
from jax.experimental import pallas as pl
from jax.experimental.pallas import tpu as pltpu
from jax.experimental.pallas import tpu_sc as plsc

MAX_LEN = 8192
EMB = 64

NC = 2
NS = 16
NW = NC * NS
LANES = 16


def _fuse_body(w_ref, p_ref, o_ref):
    o_ref[...] = w_ref[...] + p_ref[...]


def _fuse_tables(word_head, pos_table):
    return pl.pallas_call(
        _fuse_body,
        out_shape=jax.ShapeDtypeStruct((MAX_LEN, EMB), jnp.float32),
    )(word_head, pos_table)


def _gather_kernel(B, T):
    assert B % NW == 0
    SLAB = B // NW
    assert SLAB % LANES == 0
    NJ = SLAB // LANES
    assert T % 2 == 0
    pairs = T // 2

    mesh = plsc.VectorSubcoreMesh(core_axis_name="c", subcore_axis_name="s")

    TPAD = SLAB + 1

    @functools.partial(
        pl.kernel,
        out_type=jax.ShapeDtypeStruct((T * 8, NW, 8, SLAB), jnp.float32),
        mesh=mesh,
        scratch_types=[
            pltpu.VMEM((SLAB, T), jnp.int32),
            pltpu.VMEM((T, SLAB), jnp.int32),
            pltpu.VMEM((SLAB, EMB), jnp.float32),
            pltpu.VMEM((SLAB, EMB), jnp.float32),
            pltpu.VMEM((8, 8, TPAD), jnp.float32),
            pltpu.VMEM((8, 8, TPAD), jnp.float32),
            pltpu.SemaphoreType.DMA,
            pltpu.SemaphoreType.DMA,
            pltpu.SemaphoreType.DMA,
            pltpu.SemaphoreType.DMA,
        ],
        compiler_params=pltpu.CompilerParams(use_tc_tiling_on_sc=False,
                                             needs_layout_passes=False),
    )
    def k(idx_hbm, table_hbm, out_hbm, idx_raw, idx_t, gbuf_a, gbuf_b,
          tbuf_a, tbuf_b, gsem_a, gsem_b, osem_a, osem_b):
        wid = lax.axis_index("s") * NC + lax.axis_index("c")
        b0 = wid * SLAB
        pltpu.sync_copy(idx_hbm.at[pl.ds(b0, SLAB)], idx_raw)

        iota = lax.iota(jnp.int32, LANES)
        rows = [iota + (LANES * j) for j in range(NJ)]

        def idx_transpose(t, _):
            tv = jnp.full((LANES,), t, jnp.int32)
            for j in range(NJ):
                v = plsc.load_gather(idx_raw, [rows[j], tv])
                plsc.store_scatter(idx_t, [tv, rows[j]], v)
            return 0

        lax.fori_loop(0, T, idx_transpose, 0)

        def fire_gather(gbuf, t, sem):
            pltpu.async_copy(table_hbm.at[idx_t.at[t]], gbuf, sem)

        def wait_gather(gbuf, sem):
            pltpu.make_async_copy(table_hbm.at[pl.ds(0, SLAB)], gbuf,
                                  sem).wait()

        NK = EMB // LANES
        erows = [iota + (LANES * k) for k in range(NK)]
        ehis = [(iota + LANES * k) // 8 for k in range(NK)]
        elos = [(iota + LANES * k) % 8 for k in range(NK)]

        def transpose(gbuf, tbuf):
            @plsc.parallel_loop(0, SLAB // 8, unroll=2)
            def chunk(i):
                b_base = i * 8
                for bb in range(8):
                    b = b_base + bb
                    bv = jnp.full((LANES,), b, jnp.int32)
                    for k in range(NK):
                        v = gbuf[b, pl.ds(LANES * k, LANES)]
                        plsc.store_scatter(tbuf, [ehis[k], elos[k], bv], v)

        def out_copy(tbuf, t, sem):
            return pltpu.make_async_copy(
                tbuf.at[:, :, pl.ds(0, SLAB)],
                out_hbm.at[pl.ds(t * 8, 8), wid], sem)

        fire_gather(gbuf_a, 0, gsem_a)
        fire_gather(gbuf_b, 1, gsem_b)

        def pair(i, _):
            t_a = 2 * i
            t_b = 2 * i + 1

            wait_gather(gbuf_a, gsem_a)

            @pl.when(i > 0)
            def _():
                out_copy(tbuf_a, t_a - 2, osem_a).wait()

            transpose(gbuf_a, tbuf_a)
            out_copy(tbuf_a, t_a, osem_a).start()

            @pl.when(i < pairs - 1)
            def _():
                fire_gather(gbuf_a, t_a + 2, gsem_a)

            wait_gather(gbuf_b, gsem_b)

            @pl.when(i > 0)
            def _():
                out_copy(tbuf_b, t_b - 2, osem_b).wait()

            transpose(gbuf_b, tbuf_b)
            out_copy(tbuf_b, t_b, osem_b).start()

            @pl.when(i < pairs - 1)
            def _():
                fire_gather(gbuf_b, t_b + 2, gsem_b)

            return 0

        lax.fori_loop(0, pairs, pair, 0)
        out_copy(tbuf_a, T - 2, osem_a).wait()
        out_copy(tbuf_b, T - 1, osem_b).wait()

    return k


def kernel(X, word_table, pos_table):
    B, T = X.shape
    fused = _fuse_tables(word_table[:MAX_LEN], pos_table)
    out4 = _gather_kernel(B, T)(X, fused)
    y5 = out4.reshape(T, 8, NW, 8, B // NW)
    return jnp.transpose(y5, (2, 4, 0, 1, 3)).reshape(B, T, EMB)
